# baseline (speedup 1.0000x reference)
import functools

import jax
import jax.numpy as jnp
from jax import lax
from jax.experimental import pallas as pl
from jax.experimental.pallas import tpu as pltpu
from jax.experimental.pallas import tpu_sc as plsc

_NUM_CORES = 2
_NUM_SUBCORES = 16
_NW = _NUM_CORES * _NUM_SUBCORES

_SEQ = 50
_SEQ_PAD = 56
_BB = 8


def _tc_project_table(table, w_t):
    V, D = table.shape
    P = w_t.shape[1]
    BM = 8192

    def mm(a_ref, w_ref, o_ref):
        o_ref[...] = jnp.dot(a_ref[...], w_ref[...],
                             preferred_element_type=jnp.float32)

    return pl.pallas_call(
        mm,
        grid=(pl.cdiv(V, BM),),
        in_specs=[
            pl.BlockSpec((BM, D), lambda i: (i, 0)),
            pl.BlockSpec((D, P), lambda i: (0, 0)),
        ],
        out_specs=pl.BlockSpec((BM, P), lambda i: (i, 0)),
        out_shape=jax.ShapeDtypeStruct((V, P), jnp.float32),
    )(table, w_t)


def _sc_gather_out(ptab, idx_pad, n_batch):
    P = ptab.shape[1]
    b_per_w = n_batch // _NW
    n_chunks = b_per_w // _BB
    idx_per_w = b_per_w * _SEQ_PAD
    rows_per_chunk = _BB * _SEQ_PAD
    mesh = plsc.VectorSubcoreMesh(core_axis_name="c", subcore_axis_name="s")

    @functools.partial(
        pl.kernel,
        out_type=jax.ShapeDtypeStruct((n_batch * _SEQ_PAD, P), jnp.float32),
        mesh=mesh,
        scratch_types=[
            pltpu.VMEM((rows_per_chunk,), jnp.int32),
            pltpu.VMEM((rows_per_chunk, P), jnp.float32),
            pltpu.SemaphoreType.DMA,
        ],
        compiler_params=pltpu.CompilerParams(use_tc_tiling_on_sc=True),
    )
    def gather_kernel(ptab_hbm, idx_hbm, out_hbm, idx_c, rows_v, sem):
        wid = lax.axis_index("s") * _NUM_CORES + lax.axis_index("c")
        row0 = wid * idx_per_w

        def body(k, carry):
            pltpu.sync_copy(
                idx_hbm.at[pl.ds(row0 + k * rows_per_chunk, rows_per_chunk)],
                idx_c)
            pltpu.async_copy(ptab_hbm.at[idx_c], rows_v, sem).wait()
            pltpu.sync_copy(
                rows_v, out_hbm.at[pl.ds(row0 + k * rows_per_chunk,
                                         rows_per_chunk)])
            return carry

        lax.fori_loop(0, n_chunks, body, 0)

    return gather_kernel(ptab, idx_pad)


def kernel(x, emb_table, proj_w):
    bsz, seq = x.shape
    ptab = _tc_project_table(emb_table, proj_w.T)
    x_pad = jnp.pad(x, ((0, 0), (0, _SEQ_PAD - seq)))
    idx = x_pad.reshape(-1)
    flat = _sc_gather_out(ptab, idx, bsz)
    return flat.reshape(bsz, _SEQ_PAD, proj_w.shape[0])[:, :seq, :]


# speedup vs baseline: 1.0005x; 1.0005x over previous
import functools

import jax
import jax.numpy as jnp
from jax import lax
from jax.experimental import pallas as pl
from jax.experimental.pallas import tpu as pltpu
from jax.experimental.pallas import tpu_sc as plsc

_NUM_CORES = 2
_NUM_SUBCORES = 16
_NW = _NUM_CORES * _NUM_SUBCORES

_SEQ = 50
_SEQ_PAD = 56
_BB = 8


def _tc_project_table(table, w_t):
    V, D = table.shape
    P = w_t.shape[1]
    BM = 8192

    def mm(a_ref, w_ref, o_ref):
        o_ref[...] = jnp.dot(a_ref[...], w_ref[...],
                             preferred_element_type=jnp.float32)

    return pl.pallas_call(
        mm,
        grid=(pl.cdiv(V, BM),),
        in_specs=[
            pl.BlockSpec((BM, D), lambda i: (i, 0)),
            pl.BlockSpec((D, P), lambda i: (0, 0)),
        ],
        out_specs=pl.BlockSpec((BM, P), lambda i: (i, 0)),
        out_shape=jax.ShapeDtypeStruct((V, P), jnp.float32),
    )(table, w_t)


def _sc_gather_out(ptab, idx_pad, n_batch):
    P = ptab.shape[1]
    b_per_w = n_batch // _NW
    n_chunks = b_per_w // _BB
    idx_per_w = b_per_w * _SEQ_PAD
    rows_per_chunk = _BB * _SEQ_PAD
    mesh = plsc.VectorSubcoreMesh(core_axis_name="c", subcore_axis_name="s")

    @functools.partial(
        pl.kernel,
        out_type=jax.ShapeDtypeStruct((n_batch * _SEQ_PAD, P), jnp.float32),
        mesh=mesh,
        scratch_types=[
            pltpu.VMEM((rows_per_chunk,), jnp.int32),
            pltpu.VMEM((rows_per_chunk, P), jnp.float32),
            pltpu.SemaphoreType.DMA,
        ],
        compiler_params=pltpu.CompilerParams(use_tc_tiling_on_sc=False),
    )
    def gather_kernel(ptab_hbm, idx_hbm, out_hbm, idx_c, rows_v, sem):
        wid = lax.axis_index("s") * _NUM_CORES + lax.axis_index("c")
        row0 = wid * idx_per_w

        def body(k, carry):
            pltpu.sync_copy(
                idx_hbm.at[pl.ds(row0 + k * rows_per_chunk, rows_per_chunk)],
                idx_c)
            pltpu.async_copy(ptab_hbm.at[idx_c], rows_v, sem).wait()
            pltpu.sync_copy(
                rows_v, out_hbm.at[pl.ds(row0 + k * rows_per_chunk,
                                         rows_per_chunk)])
            return carry

        lax.fori_loop(0, n_chunks, body, 0)

    return gather_kernel(ptab, idx_pad)


def kernel(x, emb_table, proj_w):
    bsz, seq = x.shape
    ptab = _tc_project_table(emb_table, proj_w.T)
    x_pad = jnp.pad(x, ((0, 0), (0, _SEQ_PAD - seq)))
    idx = x_pad.reshape(-1)
    flat = _sc_gather_out(ptab, idx, bsz)
    return flat.reshape(bsz, _SEQ_PAD, proj_w.shape[0])[:, :seq, :]


# speedup vs baseline: 3.0539x; 3.0522x over previous
import functools

import jax
import jax.numpy as jnp
from jax import lax
from jax.experimental import pallas as pl
from jax.experimental.pallas import tpu as pltpu
from jax.experimental.pallas import tpu_sc as plsc

_NUM_CORES = 2
_NUM_SUBCORES = 16
_NW = _NUM_CORES * _NUM_SUBCORES

_CHUNK = 1024
_SEQ = 50
_BB = 32


def _sc_gather(table, idx_flat):
    B = idx_flat.shape[0]
    D = table.shape[1]
    b_per_w = B // _NW
    n_chunks = b_per_w // _CHUNK
    mesh = plsc.VectorSubcoreMesh(core_axis_name="c", subcore_axis_name="s")

    @functools.partial(
        pl.kernel,
        out_type=jax.ShapeDtypeStruct((B, D), jnp.float32),
        mesh=mesh,
        scratch_types=[
            pltpu.VMEM((_CHUNK,), jnp.int32),
            pltpu.VMEM((_CHUNK, D), jnp.float32),
            pltpu.SemaphoreType.DMA,
        ],
        compiler_params=pltpu.CompilerParams(use_tc_tiling_on_sc=False),
    )
    def gather_kernel(table_hbm, idx_hbm, out_hbm, idx_c, rows_v, sem):
        wid = lax.axis_index("s") * _NUM_CORES + lax.axis_index("c")
        base = wid * b_per_w

        def body(k, carry):
            off = base + k * _CHUNK
            pltpu.sync_copy(idx_hbm.at[pl.ds(off, _CHUNK)], idx_c)
            pltpu.async_copy(table_hbm.at[idx_c], rows_v, sem).wait()
            pltpu.sync_copy(rows_v, out_hbm.at[pl.ds(off, _CHUNK)])
            return carry

        lax.fori_loop(0, n_chunks, body, 0)

    return gather_kernel(table, idx_flat)


def _tc_project(gathered, w_t, bsz):
    D = gathered.shape[1]
    P = w_t.shape[1]
    rows_per_blk = _BB * _SEQ

    def mm(a_ref, w_ref, o_ref):
        t = jnp.dot(a_ref[...], w_ref[...],
                    preferred_element_type=jnp.float32)
        o_ref[...] = t.reshape(_BB, _SEQ, P)

    return pl.pallas_call(
        mm,
        grid=(bsz // _BB,),
        in_specs=[
            pl.BlockSpec((rows_per_blk, D), lambda i: (i, 0)),
            pl.BlockSpec((D, P), lambda i: (0, 0)),
        ],
        out_specs=pl.BlockSpec((_BB, _SEQ, P), lambda i: (i, 0, 0)),
        out_shape=jax.ShapeDtypeStruct((bsz, _SEQ, P), jnp.float32),
    )(gathered, w_t)


def kernel(x, emb_table, proj_w):
    bsz, seq = x.shape
    d_emb = emb_table.shape[1]
    idx = x.reshape(-1)
    g = _sc_gather(emb_table, idx)
    return _tc_project(g, proj_w.T, bsz)


# speedup vs baseline: 9.4317x; 3.0885x over previous
import functools

import jax
import jax.numpy as jnp
from jax import lax
from jax.experimental import pallas as pl
from jax.experimental.pallas import tpu as pltpu
from jax.experimental.pallas import tpu_sc as plsc

_NUM_CORES = 2
_NUM_SUBCORES = 16
_NW = _NUM_CORES * _NUM_SUBCORES

_CHUNK = 640


def _tc_project_table(table_t, w_t):
    D, V = table_t.shape
    P = w_t.shape[1]
    BM = 8192

    def mm(a_ref, w_ref, o_ref):
        o_ref[...] = lax.dot_general(
            a_ref[...], w_ref[...],
            dimension_numbers=(((0,), (0,)), ((), ())),
            preferred_element_type=jnp.float32)

    return pl.pallas_call(
        mm,
        grid=(pl.cdiv(V, BM),),
        in_specs=[
            pl.BlockSpec((D, BM), lambda i: (0, i)),
            pl.BlockSpec((D, P), lambda i: (0, 0)),
        ],
        out_specs=pl.BlockSpec((BM, P), lambda i: (i, 0)),
        out_shape=jax.ShapeDtypeStruct((V, P), jnp.float32),
    )(table_t, w_t)


def _sc_gather(ptab, idx_flat):
    B = idx_flat.shape[0]
    P = ptab.shape[1]
    b_per_w = B // _NW
    n_chunks = b_per_w // _CHUNK
    mesh = plsc.VectorSubcoreMesh(core_axis_name="c", subcore_axis_name="s")

    @functools.partial(
        pl.kernel,
        out_type=jax.ShapeDtypeStruct((B, P), jnp.float32),
        mesh=mesh,
        scratch_types=[
            pltpu.VMEM((_CHUNK,), jnp.int32),
            pltpu.VMEM((_CHUNK, P), jnp.float32),
            pltpu.SemaphoreType.DMA,
        ],
        compiler_params=pltpu.CompilerParams(use_tc_tiling_on_sc=False),
    )
    def gather_kernel(ptab_hbm, idx_hbm, out_hbm, idx_c, rows_v, sem):
        wid = lax.axis_index("s") * _NUM_CORES + lax.axis_index("c")
        base = wid * b_per_w

        def body(k, carry):
            off = base + k * _CHUNK
            pltpu.sync_copy(idx_hbm.at[pl.ds(off, _CHUNK)], idx_c)
            pltpu.async_copy(ptab_hbm.at[idx_c], rows_v, sem).wait()
            pltpu.sync_copy(rows_v, out_hbm.at[pl.ds(off, _CHUNK)])
            return carry

        lax.fori_loop(0, n_chunks, body, 0)

    return gather_kernel(ptab, idx_flat)


def kernel(x, emb_table, proj_w):
    bsz, seq = x.shape
    ptab = _tc_project_table(emb_table.T, proj_w.T)
    idx_t = x.T.reshape(-1)
    flat = _sc_gather(ptab, idx_t)
    return flat.reshape(seq, bsz, proj_w.shape[0]).transpose(1, 0, 2)


# speedup vs baseline: 9.8896x; 1.0485x over previous
import functools

import jax
import jax.numpy as jnp
from jax import lax
from jax.experimental import pallas as pl
from jax.experimental.pallas import tpu as pltpu
from jax.experimental.pallas import tpu_sc as plsc

_NUM_CORES = 2
_NUM_SUBCORES = 16
_NW = _NUM_CORES * _NUM_SUBCORES

_CHUNK = 400


def _tc_project_table(table_t, w_t):
    D, V = table_t.shape
    P = w_t.shape[1]
    BM = 8192

    def mm(a_ref, w_ref, o_ref):
        o_ref[...] = lax.dot_general(
            a_ref[...], w_ref[...],
            dimension_numbers=(((0,), (0,)), ((), ())),
            preferred_element_type=jnp.float32)

    return pl.pallas_call(
        mm,
        grid=(pl.cdiv(V, BM),),
        in_specs=[
            pl.BlockSpec((D, BM), lambda i: (0, i)),
            pl.BlockSpec((D, P), lambda i: (0, 0)),
        ],
        out_specs=pl.BlockSpec((BM, P), lambda i: (i, 0)),
        out_shape=jax.ShapeDtypeStruct((V, P), jnp.float32),
    )(table_t, w_t)


def _sc_gather(ptab, idx_flat):
    B = idx_flat.shape[0]
    P = ptab.shape[1]
    b_per_w = B // _NW
    n_chunks = b_per_w // _CHUNK
    mesh = plsc.VectorSubcoreMesh(core_axis_name="c", subcore_axis_name="s")

    @functools.partial(
        pl.kernel,
        out_type=jax.ShapeDtypeStruct((B, P), jnp.float32),
        mesh=mesh,
        scratch_types=[
            pltpu.VMEM((_CHUNK,), jnp.int32),
            pltpu.VMEM((_CHUNK,), jnp.int32),
            pltpu.VMEM((_CHUNK, P), jnp.float32),
            pltpu.VMEM((_CHUNK, P), jnp.float32),
            pltpu.SemaphoreType.DMA,
            pltpu.SemaphoreType.DMA,
        ],
        compiler_params=pltpu.CompilerParams(use_tc_tiling_on_sc=False),
    )
    def gather_kernel(ptab_hbm, idx_hbm, out_hbm,
                      idx_a, idx_b, rows_a, rows_b, sem_a, sem_b):
        wid = lax.axis_index("s") * _NUM_CORES + lax.axis_index("c")
        base = wid * b_per_w
        idx_bufs = (idx_a, idx_b)
        row_bufs = (rows_a, rows_b)
        sems = (sem_a, sem_b)

        pltpu.sync_copy(idx_hbm.at[pl.ds(base, _CHUNK)], idx_a)
        pltpu.async_copy(ptab_hbm.at[idx_a], rows_a, sem_a)

        def body(j, carry):
            for b in (0, 1):
                k = 2 * j + b
                q = 1 - b

                @pl.when(k + 1 < n_chunks)
                def _launch_next():
                    off_n = base + (k + 1) * _CHUNK
                    pltpu.sync_copy(idx_hbm.at[pl.ds(off_n, _CHUNK)],
                                    idx_bufs[q])
                    pltpu.async_copy(ptab_hbm.at[idx_bufs[q]],
                                     row_bufs[q], sems[q])

                pltpu.make_async_copy(ptab_hbm.at[idx_bufs[b]],
                                      row_bufs[b], sems[b]).wait()
                off = base + k * _CHUNK
                pltpu.sync_copy(row_bufs[b], out_hbm.at[pl.ds(off, _CHUNK)])
            return carry

        lax.fori_loop(0, n_chunks // 2, body, 0)

    return gather_kernel(ptab, idx_flat)


def kernel(x, emb_table, proj_w):
    bsz, seq = x.shape
    ptab = _tc_project_table(emb_table.T, proj_w.T)
    idx_t = x.T.reshape(-1)
    flat = _sc_gather(ptab, idx_t)
    return flat.reshape(seq, bsz, proj_w.shape[0]).transpose(1, 0, 2)


# speedup vs baseline: 10.3257x; 1.0441x over previous
import functools

import jax
import jax.numpy as jnp
from jax import lax
from jax.experimental import pallas as pl
from jax.experimental.pallas import tpu as pltpu
from jax.experimental.pallas import tpu_sc as plsc

_NUM_CORES = 2
_NUM_SUBCORES = 16
_NW = _NUM_CORES * _NUM_SUBCORES

_CHUNK = 400


def _tc_project_table(table_t, w_t):
    D, V = table_t.shape
    P = w_t.shape[1]
    BM = 16384

    def mm(a_ref, w_ref, o_ref):
        o_ref[...] = lax.dot_general(
            a_ref[...], w_ref[...],
            dimension_numbers=(((0,), (0,)), ((), ())),
            preferred_element_type=jnp.float32)

    return pl.pallas_call(
        mm,
        grid=(pl.cdiv(V, BM),),
        in_specs=[
            pl.BlockSpec((D, BM), lambda i: (0, i)),
            pl.BlockSpec((D, P), lambda i: (0, 0)),
        ],
        out_specs=pl.BlockSpec((BM, P), lambda i: (i, 0)),
        out_shape=jax.ShapeDtypeStruct((V, P), jnp.float32),
    )(table_t, w_t)


def _sc_gather(ptab, idx_flat):
    B = idx_flat.shape[0]
    P = ptab.shape[1]
    b_per_w = B // _NW
    n_chunks = b_per_w // _CHUNK
    mesh = plsc.VectorSubcoreMesh(core_axis_name="c", subcore_axis_name="s")

    @functools.partial(
        pl.kernel,
        out_type=jax.ShapeDtypeStruct((B, P), jnp.float32),
        mesh=mesh,
        scratch_types=[
            pltpu.VMEM((_CHUNK,), jnp.int32),
            pltpu.VMEM((_CHUNK,), jnp.int32),
            pltpu.VMEM((_CHUNK, P), jnp.float32),
            pltpu.VMEM((_CHUNK, P), jnp.float32),
            pltpu.SemaphoreType.DMA,
            pltpu.SemaphoreType.DMA,
        ],
        compiler_params=pltpu.CompilerParams(use_tc_tiling_on_sc=False),
    )
    def gather_kernel(ptab_hbm, idx_hbm, out_hbm,
                      idx_a, idx_b, rows_a, rows_b, sem_a, sem_b):
        wid = lax.axis_index("s") * _NUM_CORES + lax.axis_index("c")
        base = wid * b_per_w
        idx_bufs = (idx_a, idx_b)
        row_bufs = (rows_a, rows_b)
        sems = (sem_a, sem_b)

        pltpu.sync_copy(idx_hbm.at[pl.ds(base, _CHUNK)], idx_a)
        pltpu.async_copy(ptab_hbm.at[idx_a], rows_a, sem_a)

        def body(j, carry):
            for b in (0, 1):
                k = 2 * j + b
                q = 1 - b

                @pl.when(k + 1 < n_chunks)
                def _launch_next():
                    off_n = base + (k + 1) * _CHUNK
                    pltpu.sync_copy(idx_hbm.at[pl.ds(off_n, _CHUNK)],
                                    idx_bufs[q])
                    pltpu.async_copy(ptab_hbm.at[idx_bufs[q]],
                                     row_bufs[q], sems[q])

                pltpu.make_async_copy(ptab_hbm.at[idx_bufs[b]],
                                      row_bufs[b], sems[b]).wait()
                off = base + k * _CHUNK
                pltpu.sync_copy(row_bufs[b], out_hbm.at[pl.ds(off, _CHUNK)])
            return carry

        lax.fori_loop(0, n_chunks // 2, body, 0)

    return gather_kernel(ptab, idx_flat)


def kernel(x, emb_table, proj_w):
    bsz, seq = x.shape
    ptab = _tc_project_table(emb_table.T, proj_w.T)
    idx_t = x.T.reshape(-1)
    flat = _sc_gather(ptab, idx_t)
    return flat.reshape(seq, bsz, proj_w.shape[0]).transpose(1, 0, 2)


# speedup vs baseline: 10.4137x; 1.0085x over previous
import functools

import jax
import jax.numpy as jnp
from jax import lax
from jax.experimental import pallas as pl
from jax.experimental.pallas import tpu as pltpu
from jax.experimental.pallas import tpu_sc as plsc

_NUM_CORES = 2
_NUM_SUBCORES = 16
_NW = _NUM_CORES * _NUM_SUBCORES

_CHUNK = 320


def _tc_project_table(table_t, w_t):
    D, V = table_t.shape
    P = w_t.shape[1]
    BM = 32768

    def mm(a_ref, w_ref, o_ref):
        o_ref[...] = lax.dot_general(
            a_ref[...], w_ref[...],
            dimension_numbers=(((0,), (0,)), ((), ())),
            preferred_element_type=jnp.float32)

    return pl.pallas_call(
        mm,
        grid=(pl.cdiv(V, BM),),
        in_specs=[
            pl.BlockSpec((D, BM), lambda i: (0, i)),
            pl.BlockSpec((D, P), lambda i: (0, 0)),
        ],
        out_specs=pl.BlockSpec((BM, P), lambda i: (i, 0)),
        out_shape=jax.ShapeDtypeStruct((V, P), jnp.float32),
    )(table_t, w_t)


def _sc_gather(ptab, idx_flat):
    B = idx_flat.shape[0]
    P = ptab.shape[1]
    b_per_w = B // _NW
    n_chunks = b_per_w // _CHUNK
    mesh = plsc.VectorSubcoreMesh(core_axis_name="c", subcore_axis_name="s")

    @functools.partial(
        pl.kernel,
        out_type=jax.ShapeDtypeStruct((B, P), jnp.float32),
        mesh=mesh,
        scratch_types=[
            pltpu.VMEM((_CHUNK,), jnp.int32),
            pltpu.VMEM((_CHUNK,), jnp.int32),
            pltpu.VMEM((_CHUNK,), jnp.int32),
            pltpu.VMEM((_CHUNK, P), jnp.float32),
            pltpu.VMEM((_CHUNK, P), jnp.float32),
            pltpu.VMEM((_CHUNK, P), jnp.float32),
            pltpu.SemaphoreType.DMA,
            pltpu.SemaphoreType.DMA,
            pltpu.SemaphoreType.DMA,
        ],
        compiler_params=pltpu.CompilerParams(use_tc_tiling_on_sc=False),
    )
    def gather_kernel(ptab_hbm, idx_hbm, out_hbm,
                      idx_a, idx_b, idx_c, rows_a, rows_b, rows_c,
                      sem_a, sem_b, sem_c):
        wid = lax.axis_index("s") * _NUM_CORES + lax.axis_index("c")
        base = wid * b_per_w
        idx_bufs = (idx_a, idx_b, idx_c)
        row_bufs = (rows_a, rows_b, rows_c)
        sems = (sem_a, sem_b, sem_c)

        pltpu.sync_copy(idx_hbm.at[pl.ds(base, _CHUNK)], idx_a)
        pltpu.async_copy(ptab_hbm.at[idx_a], rows_a, sem_a)
        pltpu.sync_copy(idx_hbm.at[pl.ds(base + _CHUNK, _CHUNK)], idx_b)
        pltpu.async_copy(ptab_hbm.at[idx_b], rows_b, sem_b)

        def body(j, carry):
            for b in (0, 1, 2):
                k = 3 * j + b
                q = (b + 2) % 3

                @pl.when(k + 2 < n_chunks)
                def _launch_ahead():
                    off_n = base + (k + 2) * _CHUNK
                    pltpu.sync_copy(idx_hbm.at[pl.ds(off_n, _CHUNK)],
                                    idx_bufs[q])
                    pltpu.async_copy(ptab_hbm.at[idx_bufs[q]],
                                     row_bufs[q], sems[q])

                @pl.when(k < n_chunks)
                def _drain():
                    pltpu.make_async_copy(ptab_hbm.at[idx_bufs[b]],
                                          row_bufs[b], sems[b]).wait()
                    off = base + k * _CHUNK
                    pltpu.sync_copy(row_bufs[b],
                                    out_hbm.at[pl.ds(off, _CHUNK)])
            return carry

        lax.fori_loop(0, pl.cdiv(n_chunks, 3), body, 0)

    return gather_kernel(ptab, idx_flat)


def kernel(x, emb_table, proj_w):
    bsz, seq = x.shape
    ptab = _tc_project_table(emb_table.T, proj_w.T)
    idx_t = x.T.reshape(-1)
    flat = _sc_gather(ptab, idx_t)
    return flat.reshape(seq, bsz, proj_w.shape[0]).transpose(1, 0, 2)
